# baseline (speedup 1.0000x reference)
import functools

import jax
import jax.numpy as jnp
from jax import lax
from jax.experimental import pallas as pl
from jax.experimental.pallas import tpu as pltpu

_SMOOTH = 0.1
_NBUF = 3


def _chunks(C):
    full, rem = divmod(C, 128)
    out = [(k * 128, None) for k in range(full)]
    if rem:
        out.append((C - 128, full * 128))
    return out


def _ohem_kernel(x_hbm, t_ref, o_ref, xbuf, ps_ref, sem, *, nblocks, keep):
    i = pl.program_id(0)
    NB, R, C = xbuf.shape

    def start(blk):
        slot = lax.rem(blk, NB)
        pltpu.make_async_copy(
            x_hbm.at[pl.ds(blk * R, R), :], xbuf.at[slot], sem.at[slot]
        ).start()

    @pl.when(i == 0)
    def _prologue():
        for j in range(_NBUF):
            start(jnp.int32(j))

    slot = lax.rem(i, NB)
    pltpu.make_async_copy(
        x_hbm.at[pl.ds(i * R, R), :], xbuf.at[slot], sem.at[slot]
    ).wait()

    t = t_ref[0, 0, :]
    tcol = t[:, None]
    m = jnp.full((R, 128), -3.0e38, dtype=jnp.float32)
    sx = jnp.zeros((R, 128), dtype=jnp.float32)
    xt = jnp.zeros((R, 128), dtype=jnp.float32)
    for off, mask_from in _chunks(C):
        xc = xbuf[slot, :, off:off + 128]
        cols = lax.broadcasted_iota(jnp.int32, (R, 128), 1) + off
        hit = cols == tcol
        if mask_from is not None:
            valid = cols >= mask_from
            xc = jnp.where(valid, xc, 0.0)
            m = jnp.maximum(m, jnp.where(valid, xc, -3.0e38))
            hit = hit & valid
        else:
            m = jnp.maximum(m, xc)
        sx = sx + xc
        xt = xt + jnp.where(hit, xc, 0.0)
    mrow = jnp.max(m, axis=1, keepdims=True)
    s_row = jnp.sum(sx, axis=1)
    xt_row = jnp.sum(xt, axis=1)

    e = jnp.zeros((R, 128), dtype=jnp.float32)
    for off, mask_from in _chunks(C):
        xc = xbuf[slot, :, off:off + 128]
        if mask_from is not None:
            cols = lax.broadcasted_iota(jnp.int32, (R, 128), 1) + off
            xc = jnp.where(cols >= mask_from, xc, -3.0e38)
        e = e + jnp.exp(xc - mrow)
    lse = jnp.log(jnp.sum(e, axis=1)) + mrow[:, 0]

    a = 1.0 - _SMOOTH - _SMOOTH / (C - 1)
    b = _SMOOTH / (C - 1)
    ps_ref[i, :] = lse - a * xt_row - b * s_row

    @pl.when(i + _NBUF < nblocks)
    def _refill():
        start(i + _NBUF)

    @pl.when(i == nblocks - 1)
    def _select():
        v = ps_ref[...]
        bits = lax.bitcast_convert_type(v, jnp.int32)
        skey = jnp.where(bits < 0, bits ^ jnp.int32(0x7FFFFFFF), bits)

        def body(j, prefix):
            cand = prefix + (jnp.int32(1) << jnp.int32(31 - j))
            cnt = jnp.sum((skey >= cand).astype(jnp.int32))
            return jnp.where(cnt >= keep, cand, prefix)

        kth = lax.fori_loop(0, 32, body, jnp.int32(-2147483648))
        tau_bits = jnp.where(kth < 0, kth ^ jnp.int32(0x7FFFFFFF), kth)
        tau = lax.bitcast_convert_type(tau_bits, jnp.float32)
        gt = skey > kth
        cnt_gt = jnp.sum(gt.astype(jnp.int32))
        sum_gt = jnp.sum(jnp.where(gt, v, 0.0))
        total = sum_gt + (keep - cnt_gt).astype(jnp.float32) * tau
        o_ref[...] = jnp.reshape(total / keep, (1, 1))


def kernel(input, target):
    B, C = input.shape
    R = 2048
    G = B // R
    keep = min(B, int(B * 0.7))
    t3 = target.astype(jnp.int32).reshape(G, 1, R)
    out = pl.pallas_call(
        functools.partial(_ohem_kernel, nblocks=G, keep=keep),
        grid=(G,),
        in_specs=[
            pl.BlockSpec(memory_space=pl.ANY),
            pl.BlockSpec((1, 1, R), lambda i: (i, 0, 0)),
        ],
        out_specs=pl.BlockSpec((1, 1), lambda i: (0, 0)),
        out_shape=jax.ShapeDtypeStruct((1, 1), jnp.float32),
        scratch_shapes=[
            pltpu.VMEM((_NBUF, R, C), jnp.float32),
            pltpu.VMEM((G, R), jnp.float32),
            pltpu.SemaphoreType.DMA((_NBUF,)),
        ],
    )(input, t3)
    return out[0, 0]
